# baseline (speedup 1.0000x reference)
import functools

import jax
import jax.numpy as jnp
from jax import lax
from jax.experimental import pallas as pl
from jax.experimental.pallas import tpu as pltpu
from jax.experimental.pallas import tpu_sc as plsc

M = 100000
D = 64
B = 16384

NS = 16
NC = 2
NT = NS * NC
L = 16
R = 256
CR = NT * R
NFULL = M // CR
TAILROWS = M - NFULL * CR
TFULL = TAILROWS // R
TREM = TAILROWS - TFULL * R
SW = 1024
NSTRIP = B // SW
SEG = 128

_mesh = plsc.VectorSubcoreMesh(core_axis_name="c", subcore_axis_name="s")


@functools.partial(
    pl.kernel,
    out_type=jax.ShapeDtypeStruct((M, D), jnp.float32),
    mesh=_mesh,
    compiler_params=pltpu.CompilerParams(
        needs_layout_passes=False, use_tc_tiling_on_sc=False),
    scratch_types=[
        pltpu.VMEM((SW,), jnp.int32),
        pltpu.VMEM((SW + L,), jnp.int32),
        pltpu.VMEM((SW + L,), jnp.int32),
        pltpu.VMEM((1, SEG), jnp.int32),
        pltpu.VMEM((SEG, D), jnp.float32),
        pltpu.VMEM((R, D), jnp.float32),
        pltpu.VMEM_SHARED((B,), jnp.int32),
        pltpu.VMEM_SHARED((B, D), jnp.float32),
    ],
)
def _scatter_add_sc(mem_hbm, idx_hbm, val_hbm, out_hbm,
                    idx_v, pos1_v, lrow1_v, pos_row, stage, accum,
                    idx_sh, val_sh):
    c = lax.axis_index("c")
    s = lax.axis_index("s")
    t = c * NS + s
    bpt = B // NS

    pltpu.sync_copy(idx_hbm.at[pl.ds(s * bpt, bpt)],
                    idx_sh.at[pl.ds(s * bpt, bpt)])
    pltpu.sync_copy(val_hbm.at[pl.ds(s * bpt, bpt)],
                    val_sh.at[pl.ds(s * bpt, bpt)])
    plsc.subcore_barrier()

    lane = lax.iota(jnp.int32, L)
    ones = jnp.full((L,), 1, jnp.int32)
    zeros = jnp.zeros((L,), jnp.int32)

    def do_chunk(k, nrows):
        lo_t = k * CR + t * R
        pltpu.sync_copy(mem_hbm.at[pl.ds(lo_t, nrows)],
                        accum.at[pl.ds(0, nrows)])
        hi_t = lo_t + nrows

        def strip_body(u, carry0):
            pltpu.sync_copy(idx_sh.at[pl.ds(u * SW, SW)], idx_v)

            def scan_body(g, n):
                v = idx_v[pl.ds(g * L, L)]
                inr = (v >= lo_t) & (v < hi_t)
                inr_i = jnp.where(inr, ones, zeros)
                inc = plsc.cumsum(inr_i)
                dst = inc - inr_i + n
                plsc.store_scatter(pos1_v, [dst],
                                   lane + (u * SW + g * L), mask=inr)
                plsc.store_scatter(lrow1_v, [dst], v - lo_t, mask=inr)
                return n + inc[L - 1]

            n = lax.fori_loop(0, SW // L, scan_body, jnp.int32(0))

            def seg_body(sg, carry1):
                base = sg * SEG
                for gg in range(SEG // L):
                    pv = pos1_v[pl.ds(base + gg * L, L)]
                    covered = jnp.full((L,), base + gg * L, jnp.int32) \
                        + lane < n
                    pos_row[0, pl.ds(gg * L, L)] = jnp.where(
                        covered, pv, zeros)
                pltpu.sync_copy(val_sh.at[pos_row.at[0]], stage)

                def entry_body(i, carry2):
                    lrow = lrow1_v[pl.ds(base + i, L)][0]
                    for q in range(D // L):
                        sl = pl.ds(q * L, L)
                        accum[lrow, sl] = accum[lrow, sl] + stage[i, sl]
                    return carry2

                nseg = jnp.minimum(n - base, SEG)
                lax.fori_loop(0, nseg, entry_body, jnp.int32(0))
                return carry1

            lax.fori_loop(0, (n + SEG - 1) // SEG, seg_body, jnp.int32(0))
            return carry0

        lax.fori_loop(0, NSTRIP, strip_body, jnp.int32(0))
        pltpu.sync_copy(accum.at[pl.ds(0, nrows)],
                        out_hbm.at[pl.ds(lo_t, nrows)])

    def chunk_body(k, carry):
        do_chunk(k, R)
        return carry

    lax.fori_loop(0, NFULL, chunk_body, jnp.int32(0))

    @pl.when(t < TFULL)
    def _():
        do_chunk(jnp.int32(NFULL), R)

    @pl.when(t == TFULL)
    def _():
        do_chunk(jnp.int32(NFULL), TREM)


def kernel(mem, idx, val):
    return _scatter_add_sc(mem, idx.astype(jnp.int32), val)
